# Initial kernel scaffold; baseline (speedup 1.0000x reference)
import jax
import jax.numpy as jnp
from jax.experimental import pallas as pl


def kernel(nodes, neigh_idx, features, weight):
    raise NotImplementedError("write your pallas kernel here")


# speedup vs baseline: 3.5241x; 3.5241x over previous
import functools

import jax
import jax.numpy as jnp
from jax import lax
from jax.experimental import pallas as pl
from jax.experimental.pallas import tpu as pltpu
from jax.experimental.pallas import tpu_sc as plsc

N_NODES = 50000
D = 128
NUM_SAMPLE = 10

NC = 2
NS = 16
NW = NC * NS

CHUNK = 32
B_PER_W = 1568
B_PAD = B_PER_W * NW
N_CHUNKS = B_PER_W // CHUNK


def _sc_gather_body(nodes_hbm, neigh_hbm, feat_hbm, self_out, neigh_out,
                    idx_n, idx_g, self_rows, neigh_rows, neigh_acc,
                    sem1, sem2):
    wid = lax.axis_index("s") * NC + lax.axis_index("c")
    base = wid * B_PER_W

    def chunk_step(c, _):
        cb = base + c * CHUNK
        pltpu.sync_copy(nodes_hbm.at[pl.ds(cb, CHUNK)], idx_n)
        pltpu.sync_copy(neigh_hbm.at[pl.ds(cb * NUM_SAMPLE, CHUNK * NUM_SAMPLE)], idx_g)
        cp1 = pltpu.async_copy(feat_hbm.at[idx_g], neigh_rows, sem1)
        cp2 = pltpu.async_copy(feat_hbm.at[idx_n], self_rows, sem2)
        cp1.wait()
        cp2.wait()

        def reduce_node(i, _):
            r0 = i * NUM_SAMPLE
            for k in range(D // 16):
                sl = pl.ds(k * 16, 16)
                acc = neigh_rows[r0, sl]
                for j in range(1, NUM_SAMPLE):
                    acc = acc + neigh_rows[r0 + j, sl]
                neigh_acc[i, sl] = acc
            return 0

        lax.fori_loop(0, CHUNK, reduce_node, 0, unroll=False)

        pltpu.sync_copy(self_rows, self_out.at[pl.ds(cb, CHUNK)])
        pltpu.sync_copy(neigh_acc, neigh_out.at[pl.ds(cb, CHUNK)])
        return 0

    lax.fori_loop(0, N_CHUNKS, chunk_step, 0, unroll=False)


@functools.partial(jax.jit, static_argnames=())
def _sc_gather(nodes_pad, neigh_flat_pad, features):
    mesh = plsc.VectorSubcoreMesh(core_axis_name="c", subcore_axis_name="s")
    fn = pl.kernel(
        _sc_gather_body,
        out_type=[
            jax.ShapeDtypeStruct((B_PAD, D), jnp.float32),
            jax.ShapeDtypeStruct((B_PAD, D), jnp.float32),
        ],
        mesh=mesh,
        scratch_types=[
            pltpu.VMEM((CHUNK,), jnp.int32),
            pltpu.VMEM((CHUNK * NUM_SAMPLE,), jnp.int32),
            pltpu.VMEM((CHUNK, D), jnp.float32),
            pltpu.VMEM((CHUNK * NUM_SAMPLE, D), jnp.float32),
            pltpu.VMEM((CHUNK, D), jnp.float32),
            pltpu.SemaphoreType.DMA,
            pltpu.SemaphoreType.DMA,
        ],
    )
    return fn(nodes_pad, neigh_flat_pad, features)


def _tc_matmul_body(w_ref, self_ref, neigh_ref, out_ref):
    w = w_ref[...]
    w1 = w[:, :D]
    w2 = w[:, D:] * jnp.float32(1.0 / NUM_SAMPLE)
    dn = (((1,), (1,)), ((), ()))
    acc = lax.dot_general(w1, self_ref[...], dn, preferred_element_type=jnp.float32)
    acc = acc + lax.dot_general(w2, neigh_ref[...], dn, preferred_element_type=jnp.float32)
    out_ref[...] = jnp.maximum(acc, 0.0)


NB = 3584


def _tc_matmul(weight, self_feats, neigh_sums):
    grid = (B_PAD // NB,)
    return pl.pallas_call(
        _tc_matmul_body,
        grid=grid,
        in_specs=[
            pl.BlockSpec((D, 2 * D), lambda i: (0, 0)),
            pl.BlockSpec((NB, D), lambda i: (i, 0)),
            pl.BlockSpec((NB, D), lambda i: (i, 0)),
        ],
        out_specs=pl.BlockSpec((D, NB), lambda i: (0, i)),
        out_shape=jax.ShapeDtypeStruct((D, B_PAD), jnp.float32),
    )(weight, self_feats, neigh_sums)


def kernel(nodes, neigh_idx, features, weight):
    nodes = nodes.astype(jnp.int32)
    neigh_flat = neigh_idx.astype(jnp.int32).reshape(-1)
    b = nodes.shape[0]
    nodes_pad = jnp.pad(nodes, (0, B_PAD - b))
    neigh_flat_pad = jnp.pad(neigh_flat, (0, (B_PAD - b) * NUM_SAMPLE))
    self_feats, neigh_sums = _sc_gather(nodes_pad, neigh_flat_pad, features)
    out = _tc_matmul(weight, self_feats, neigh_sums)
    return out[:, :b]


# speedup vs baseline: 6.3835x; 1.8114x over previous
import jax
import jax.numpy as jnp
from jax import lax
from jax.experimental import pallas as pl
from jax.experimental.pallas import tpu as pltpu
from jax.experimental.pallas import tpu_sc as plsc

N_NODES = 50000
D = 128
NUM_SAMPLE = 10

NC = 2
NS = 16
NW = NC * NS

CHUNK = 32
GCHUNK = CHUNK * NUM_SAMPLE
N_CHUNKS = 49
B_PER_W = CHUNK * N_CHUNKS


def _sc_gather_body(nodes_hbm, neigh_hbm, feat_hbm, self_out, neigh_out,
                    idxn, idxg, nra, nrb, sra, srb, acca, accb,
                    sga, sgb, soa, sob):
    wid = lax.axis_index("s") * NC + lax.axis_index("c")
    start = jnp.minimum(wid * B_PER_W, N_NODES - B_PER_W)
    pltpu.sync_copy(nodes_hbm.at[pl.ds(start, B_PER_W)], idxn)
    pltpu.sync_copy(neigh_hbm.at[pl.ds(start * NUM_SAMPLE, B_PER_W * NUM_SAMPLE)], idxg)

    def issue_gathers(c, nr, sr, sg):
        pltpu.async_copy(feat_hbm.at[idxg.at[pl.ds(c * GCHUNK, GCHUNK)]], nr, sg)
        pltpu.async_copy(feat_hbm.at[idxn.at[pl.ds(c * CHUNK, CHUNK)]], sr, sg)

    def drain_gathers(nr, sr, sg):
        pltpu.make_async_copy(feat_hbm.at[pl.ds(0, GCHUNK)], nr, sg).wait()
        pltpu.make_async_copy(feat_hbm.at[pl.ds(0, CHUNK)], sr, sg).wait()

    def reduce_chunk(nr, acc):
        def node(i, _):
            r0 = i * NUM_SAMPLE
            for k in range(D // 16):
                sl = pl.ds(k * 16, 16)
                v = nr[r0, sl]
                for j in range(1, NUM_SAMPLE):
                    v = v + nr[r0 + j, sl]
                acc[i, sl] = v
            return 0
        lax.fori_loop(0, CHUNK, node, 0, unroll=False)

    def scatter_out(c, sr, acc, so):
        cb = start + c * CHUNK
        pltpu.async_copy(sr, self_out.at[pl.ds(cb, CHUNK)], so)
        pltpu.async_copy(acc, neigh_out.at[pl.ds(cb, CHUNK)], so)

    def drain_out(sr, acc, so):
        pltpu.make_async_copy(sr, self_out.at[pl.ds(0, CHUNK)], so).wait()
        pltpu.make_async_copy(acc, neigh_out.at[pl.ds(0, CHUNK)], so).wait()

    issue_gathers(0, nra, sra, sga)

    def pair(c2, _):
        a = c2 * 2
        @pl.when(c2 > 0)
        def _():
            drain_out(srb, accb, sob)
        issue_gathers(a + 1, nrb, srb, sgb)
        drain_gathers(nra, sra, sga)
        reduce_chunk(nra, acca)
        scatter_out(a, sra, acca, soa)
        drain_out(sra, acca, soa)
        issue_gathers(a + 2, nra, sra, sga)
        drain_gathers(nrb, srb, sgb)
        reduce_chunk(nrb, accb)
        scatter_out(a + 1, srb, accb, sob)
        return 0

    lax.fori_loop(0, (N_CHUNKS - 1) // 2, pair, 0, unroll=False)

    drain_out(srb, accb, sob)
    drain_gathers(nra, sra, sga)
    reduce_chunk(nra, acca)
    scatter_out(N_CHUNKS - 1, sra, acca, soa)
    drain_out(sra, acca, soa)


def _sc_gather(nodes, neigh_flat, features):
    mesh = plsc.VectorSubcoreMesh(core_axis_name="c", subcore_axis_name="s")
    fn = pl.kernel(
        _sc_gather_body,
        out_type=[
            jax.ShapeDtypeStruct((N_NODES, D), jnp.float32),
            jax.ShapeDtypeStruct((N_NODES, D), jnp.float32),
        ],
        mesh=mesh,
        scratch_types=[
            pltpu.VMEM((B_PER_W,), jnp.int32),
            pltpu.VMEM((B_PER_W * NUM_SAMPLE,), jnp.int32),
            pltpu.VMEM((GCHUNK, D), jnp.float32),
            pltpu.VMEM((GCHUNK, D), jnp.float32),
            pltpu.VMEM((CHUNK, D), jnp.float32),
            pltpu.VMEM((CHUNK, D), jnp.float32),
            pltpu.VMEM((CHUNK, D), jnp.float32),
            pltpu.VMEM((CHUNK, D), jnp.float32),
            pltpu.SemaphoreType.DMA,
            pltpu.SemaphoreType.DMA,
            pltpu.SemaphoreType.DMA,
            pltpu.SemaphoreType.DMA,
        ],
    )
    return fn(nodes, neigh_flat, features)


def _tc_matmul_body(w_ref, self_ref, neigh_ref, out_ref):
    w = w_ref[...]
    w1 = w[:, :D]
    w2 = w[:, D:] * jnp.float32(1.0 / NUM_SAMPLE)
    dn = (((1,), (1,)), ((), ()))
    acc = lax.dot_general(w1, self_ref[...], dn, preferred_element_type=jnp.float32)
    acc = acc + lax.dot_general(w2, neigh_ref[...], dn, preferred_element_type=jnp.float32)
    out_ref[...] = jnp.maximum(acc, 0.0)


NB = 4096


def _tc_matmul(weight, self_feats, neigh_sums):
    grid = (pl.cdiv(N_NODES, NB),)
    return pl.pallas_call(
        _tc_matmul_body,
        grid=grid,
        in_specs=[
            pl.BlockSpec((D, 2 * D), lambda i: (0, 0)),
            pl.BlockSpec((NB, D), lambda i: (i, 0)),
            pl.BlockSpec((NB, D), lambda i: (i, 0)),
        ],
        out_specs=pl.BlockSpec((D, NB), lambda i: (0, i)),
        out_shape=jax.ShapeDtypeStruct((D, N_NODES), jnp.float32),
    )(weight, self_feats, neigh_sums)


def kernel(nodes, neigh_idx, features, weight):
    nodes = nodes.astype(jnp.int32)
    neigh_flat = neigh_idx.astype(jnp.int32).reshape(-1)
    self_feats, neigh_sums = _sc_gather(nodes, neigh_flat, features)
    return _tc_matmul(weight, self_feats, neigh_sums)


# speedup vs baseline: 11.5384x; 1.8075x over previous
import jax
import jax.numpy as jnp
from jax import lax
from jax.experimental import pallas as pl
from jax.experimental.pallas import tpu as pltpu
from jax.experimental.pallas import tpu_sc as plsc

N_NODES = 50000
D = 128
NUM_SAMPLE = 10

NC = 2
NS = 16
NW = NC * NS

CHUNK = 32
GCHUNK = CHUNK * NUM_SAMPLE
N_CHUNKS = 49
B_PER_W = CHUNK * N_CHUNKS


def _sc_gather_body(nodes_hbm, neigh_hbm, feat_hbm, self_out, neigh_out,
                    idxn, idxg, stga, stgb, nra, nrb, sra, srb, acca, accb,
                    sga, sgb, soa, sob):
    wid = lax.axis_index("s") * NC + lax.axis_index("c")
    start = jnp.minimum(wid * B_PER_W, N_NODES - B_PER_W)
    cps = [pltpu.async_copy(nodes_hbm.at[pl.ds(start, B_PER_W)], idxn, sga)]
    for j in range(NUM_SAMPLE):
        cps.append(pltpu.async_copy(
            neigh_hbm.at[pl.ds(j * N_NODES + start, B_PER_W)],
            idxg.at[pl.ds(j * B_PER_W, B_PER_W)], sga))
    for cp in cps:
        cp.wait()

    def stage_idx(c, stg):
        for j in range(NUM_SAMPLE):
            for k in range(CHUNK // 16):
                stg[pl.ds(j * CHUNK + k * 16, 16)] = \
                    idxg[pl.ds(j * B_PER_W + c * CHUNK + k * 16, 16)]

    def issue_gathers(c, stg, nr, sr, sg):
        stage_idx(c, stg)
        pltpu.async_copy(feat_hbm.at[stg], nr, sg)
        pltpu.async_copy(feat_hbm.at[idxn.at[pl.ds(c * CHUNK, CHUNK)]], sr, sg)

    def drain_gathers(nr, sr, sg):
        pltpu.make_async_copy(feat_hbm.at[pl.ds(0, GCHUNK)], nr, sg).wait()
        pltpu.make_async_copy(feat_hbm.at[pl.ds(0, CHUNK)], sr, sg).wait()

    def reduce_chunk(nr, acc):
        def node(i, _):
            for k in range(D // 16):
                sl = pl.ds(k * 16, 16)
                v = nr[i, sl]
                for j in range(1, NUM_SAMPLE):
                    v = v + nr[j * CHUNK + i, sl]
                acc[i, sl] = v
            return 0
        lax.fori_loop(0, CHUNK, node, 0, unroll=False)

    def scatter_out(c, sr, acc, so):
        cb = start + c * CHUNK
        pltpu.async_copy(sr, self_out.at[pl.ds(cb, CHUNK)], so)
        pltpu.async_copy(acc, neigh_out.at[pl.ds(cb, CHUNK)], so)

    def drain_out(sr, acc, so):
        pltpu.make_async_copy(sr, self_out.at[pl.ds(0, CHUNK)], so).wait()
        pltpu.make_async_copy(acc, neigh_out.at[pl.ds(0, CHUNK)], so).wait()

    issue_gathers(0, stga, nra, sra, sga)

    def pair(c2, _):
        a = c2 * 2
        @pl.when(c2 > 0)
        def _():
            drain_out(srb, accb, sob)
        issue_gathers(a + 1, stgb, nrb, srb, sgb)
        drain_gathers(nra, sra, sga)
        reduce_chunk(nra, acca)
        scatter_out(a, sra, acca, soa)
        drain_out(sra, acca, soa)
        issue_gathers(a + 2, stga, nra, sra, sga)
        drain_gathers(nrb, srb, sgb)
        reduce_chunk(nrb, accb)
        scatter_out(a + 1, srb, accb, sob)
        return 0

    lax.fori_loop(0, (N_CHUNKS - 1) // 2, pair, 0, unroll=False)

    drain_out(srb, accb, sob)
    drain_gathers(nra, sra, sga)
    reduce_chunk(nra, acca)
    scatter_out(N_CHUNKS - 1, sra, acca, soa)
    drain_out(sra, acca, soa)


def _sc_gather(nodes, neigh_flat, features):
    mesh = plsc.VectorSubcoreMesh(core_axis_name="c", subcore_axis_name="s")
    fn = pl.kernel(
        _sc_gather_body,
        out_type=[
            jax.ShapeDtypeStruct((N_NODES, D), jnp.float32),
            jax.ShapeDtypeStruct((N_NODES, D), jnp.float32),
        ],
        mesh=mesh,
        scratch_types=[
            pltpu.VMEM((B_PER_W,), jnp.int32),
            pltpu.VMEM((B_PER_W * NUM_SAMPLE,), jnp.int32),
            pltpu.VMEM((GCHUNK,), jnp.int32),
            pltpu.VMEM((GCHUNK,), jnp.int32),
            pltpu.VMEM((GCHUNK, D), jnp.float32),
            pltpu.VMEM((GCHUNK, D), jnp.float32),
            pltpu.VMEM((CHUNK, D), jnp.float32),
            pltpu.VMEM((CHUNK, D), jnp.float32),
            pltpu.VMEM((CHUNK, D), jnp.float32),
            pltpu.VMEM((CHUNK, D), jnp.float32),
            pltpu.SemaphoreType.DMA,
            pltpu.SemaphoreType.DMA,
            pltpu.SemaphoreType.DMA,
            pltpu.SemaphoreType.DMA,
        ],
    )
    return fn(nodes, neigh_flat, features)


def _tc_matmul_body(w_ref, self_ref, neigh_ref, out_ref):
    w = w_ref[...]
    w1 = w[:, :D]
    w2 = w[:, D:] * jnp.float32(1.0 / NUM_SAMPLE)
    dn = (((1,), (1,)), ((), ()))
    acc = lax.dot_general(self_ref[...], w1, dn, preferred_element_type=jnp.float32)
    acc = acc + lax.dot_general(neigh_ref[...], w2, dn, preferred_element_type=jnp.float32)
    out_ref[...] = jnp.maximum(acc, 0.0)


NB = 4096


def _tc_matmul(weight, self_feats, neigh_sums):
    grid = (pl.cdiv(N_NODES, NB),)
    return pl.pallas_call(
        _tc_matmul_body,
        grid=grid,
        in_specs=[
            pl.BlockSpec((D, 2 * D), lambda i: (0, 0)),
            pl.BlockSpec((NB, D), lambda i: (i, 0)),
            pl.BlockSpec((NB, D), lambda i: (i, 0)),
        ],
        out_specs=pl.BlockSpec((NB, D), lambda i: (i, 0)),
        out_shape=jax.ShapeDtypeStruct((N_NODES, D), jnp.float32),
    )(weight, self_feats, neigh_sums)


def kernel(nodes, neigh_idx, features, weight):
    nodes = nodes.astype(jnp.int32)
    neigh_flat = neigh_idx.astype(jnp.int32).T.reshape(-1)
    self_feats, neigh_sums = _sc_gather(nodes, neigh_flat, features)
    return _tc_matmul(weight, self_feats, neigh_sums).T
